# Initial kernel scaffold; baseline (speedup 1.0000x reference)
import jax
import jax.numpy as jnp
from jax.experimental import pallas as pl


def kernel(node_feature, edge_feature, edges_indices, params):
    raise NotImplementedError("write your pallas kernel here")


# speedup vs baseline: 2.3319x; 2.3319x over previous
import functools

import jax
import jax.numpy as jnp
from jax import lax
from jax.experimental import pallas as pl
from jax.experimental.pallas import tpu as pltpu
from jax.experimental.pallas import tpu_sc as plsc

N = 10000
E = 320000
DN = 128
DE = 16
DH = 256

NC = 2
NS = 16
NW = NC * NS

G_CH = 80
G_EPW = E // NW
G_NCH = G_EPW // G_CH

S_CH = 80
S_EPT = E // NS
S_NCH = S_EPT // S_CH
DHH = DH // 2
HALF = N // 2
TRASH = 512
ACC_ROWS = 5632
ZPT = ACC_ROWS // NS
WPT = 312

@functools.lru_cache(maxsize=None)
def _mesh():
    return plsc.VectorSubcoreMesh(
        core_axis_name="c", subcore_axis_name="s", num_cores=NC, num_subcores=NS
    )


def _sc_gather_body(x_hbm, dst3, src3, xi_out, xj_out, dst_v, src_v, bufi, bufj,
                    semi, semj):
    wid = lax.axis_index("s") * NC + lax.axis_index("c")
    base = wid * G_EPW
    pltpu.sync_copy(dst3.at[wid], dst_v)
    pltpu.sync_copy(src3.at[wid], src_v)

    def body(j, _):
        row0 = base + j * G_CH
        cpi = pltpu.async_copy(x_hbm.at[dst_v.at[j]], bufi, semi)
        cpj = pltpu.async_copy(x_hbm.at[src_v.at[j]], bufj, semj)
        cpi.wait()
        pltpu.sync_copy(bufi, xi_out.at[pl.ds(row0, G_CH)])
        cpj.wait()
        pltpu.sync_copy(bufj, xj_out.at[pl.ds(row0, G_CH)])
        return 0

    lax.fori_loop(0, G_NCH, body, 0)


@functools.lru_cache(maxsize=None)
def _sc_gather():
    return pl.kernel(
        _sc_gather_body,
        out_type=(
            jax.ShapeDtypeStruct((E, DN), jnp.float32),
            jax.ShapeDtypeStruct((E, DN), jnp.float32),
        ),
        mesh=_mesh(),
        scratch_types=[
            pltpu.VMEM((G_NCH, G_CH), jnp.int32),
            pltpu.VMEM((G_NCH, G_CH), jnp.int32),
            pltpu.VMEM((G_CH, DN), jnp.float32),
            pltpu.VMEM((G_CH, DN), jnp.float32),
            pltpu.SemaphoreType.DMA,
            pltpu.SemaphoreType.DMA,
        ],
    )


def _sc_scatter_body(msg2_hbm, dst3, agg_out, idx_v, buf, acc, sem):
    c = lax.axis_index("c")
    s = lax.axis_index("s")
    iot = lax.iota(jnp.int32, 16)
    ebase = s * S_EPT

    for p in range(2):
        pltpu.sync_copy(dst3.at[s], idx_v)
        lo = p * HALF

        def remap(j, _):
            for k in range(S_CH // 16):
                t = idx_v[j, pl.ds(k * 16, 16)]
                local = t - lo
                inb = (local >= 0) & (local < HALF)
                trash = 5120 + (((j * (S_CH // 16) + k) * 16 + iot)
                                & (TRASH - 1))
                idx_v[j, pl.ds(k * 16, 16)] = jnp.where(inb, local, trash)
            return 0

        lax.fori_loop(0, S_NCH, remap, 0)

        def zbuf(i, _):
            for k in range(DHH // 16):
                buf[i, pl.ds(k * 16, 16)] = jnp.zeros((16,), jnp.float32)
            return 0

        lax.fori_loop(0, S_CH, zbuf, 0)
        zbase = s * ZPT
        for off in (0, 80, 160, 240, ZPT - S_CH):
            pltpu.sync_copy(buf, acc.at[pl.ds(zbase + off, S_CH)])
        plsc.subcore_barrier()

        def body(j, _):
            pltpu.sync_copy(msg2_hbm.at[c, pl.ds(ebase + j * S_CH, S_CH)], buf)
            pltpu.sync_copy(buf, acc.at[idx_v.at[j]], add=True)
            return 0

        lax.fori_loop(0, S_NCH, body, 0)
        plsc.subcore_barrier()

        cnt = jnp.where(s == NS - 1, HALF - WPT * (NS - 1), WPT)
        rbase = s * WPT
        for t in range(4):
            off = jnp.minimum(jnp.int32(t * S_CH), cnt - S_CH)
            pltpu.sync_copy(acc.at[pl.ds(rbase + off, S_CH)], buf)
            pltpu.sync_copy(
                buf,
                agg_out.at[pl.ds(lo + rbase + off, S_CH),
                           pl.ds(c * DHH, DHH)],
            )
        plsc.subcore_barrier()


@functools.lru_cache(maxsize=None)
def _sc_scatter():
    return pl.kernel(
        _sc_scatter_body,
        out_type=jax.ShapeDtypeStruct((N, DH), jnp.float32),
        mesh=_mesh(),
        scratch_types=[
            pltpu.VMEM((S_NCH, S_CH), jnp.int32),
            pltpu.VMEM((S_CH, DHH), jnp.float32),
            pltpu.VMEM_SHARED((ACC_ROWS, DHH), jnp.float32),
            pltpu.SemaphoreType.DMA,
        ],
    )


BE = 2000


def _edge_mlp_body(xi, xj, e, w1a, w1b, w1c, b1, w2a, b2a, w2b, b2b, w2c, b2c,
                   msg_ref, ne_ref):
    h = (
        jnp.dot(xi[...], w1a[...], preferred_element_type=jnp.float32)
        + jnp.dot(e[...], w1b[...], preferred_element_type=jnp.float32)
        + jnp.dot(xj[...], w1c[...], preferred_element_type=jnp.float32)
        + b1[...]
    )
    h = jnp.maximum(h, 0.0)
    mi = jnp.dot(h, w2a[...], preferred_element_type=jnp.float32) + b2a[...]
    mj = jnp.dot(h, w2c[...], preferred_element_type=jnp.float32) + b2c[...]
    msg = jnp.maximum(mi, 0.0) + jnp.maximum(mj, 0.0)
    msg_ref[0] = msg[:, :DHH]
    msg_ref[1] = msg[:, DHH:]
    ne = jnp.dot(h, w2b[...], preferred_element_type=jnp.float32) + b2b[...]
    ne_ref[...] = jnp.maximum(ne, 0.0)


def _edge_mlp(xi, xj, e, w1a, w1b, w1c, b1, w2a, b2a, w2b, b2b, w2c, b2c):
    grid = (E // BE,)
    blk = lambda r, c: pl.BlockSpec((r, c), lambda i: (0, 0))
    return pl.pallas_call(
        _edge_mlp_body,
        grid=grid,
        in_specs=[
            pl.BlockSpec((BE, DN), lambda i: (i, 0)),
            pl.BlockSpec((BE, DN), lambda i: (i, 0)),
            pl.BlockSpec((BE, DE), lambda i: (i, 0)),
            blk(DN, DH), blk(DE, DH), blk(DN, DH), blk(1, DH),
            blk(DH, DH), blk(1, DH),
            blk(DH, DE), blk(1, DE),
            blk(DH, DH), blk(1, DH),
        ],
        out_specs=[
            pl.BlockSpec((2, BE, DHH), lambda i: (0, i, 0)),
            pl.BlockSpec((BE, DE), lambda i: (i, 0)),
        ],
        out_shape=[
            jax.ShapeDtypeStruct((2, E, DHH), jnp.float32),
            jax.ShapeDtypeStruct((E, DE), jnp.float32),
        ],
    )(xi, xj, e, w1a, w1b, w1c, b1, w2a, b2a, w2b, b2b, w2c, b2c)


BN = 2000


def _node_mlp_body(relu_out, agg, w3, b3, w4, b4, out_ref):
    h = jnp.dot(agg[...], w3[...], preferred_element_type=jnp.float32) + b3[...]
    h = jnp.maximum(h, 0.0)
    o = jnp.dot(h, w4[...], preferred_element_type=jnp.float32) + b4[...]
    if relu_out:
        o = jnp.maximum(o, 0.0)
    out_ref[...] = o


def _node_mlp(agg, w3, b3, w4, b4, relu_out):
    grid = (N // BN,)
    blk = lambda r, c: pl.BlockSpec((r, c), lambda i: (0, 0))
    return pl.pallas_call(
        functools.partial(_node_mlp_body, relu_out),
        grid=grid,
        in_specs=[
            pl.BlockSpec((BN, DH), lambda i: (i, 0)),
            blk(DH, DH), blk(1, DH), blk(DH, DN), blk(1, DN),
        ],
        out_specs=pl.BlockSpec((BN, DN), lambda i: (i, 0)),
        out_shape=jax.ShapeDtypeStruct((N, DN), jnp.float32),
    )(agg, w3, b3, w4, b4)


def kernel(node_feature, edge_feature, edges_indices, params):
    src = edges_indices[0]
    dst = edges_indices[1]
    dst_g = dst.reshape(NW, G_NCH, G_CH)
    src_g = src.reshape(NW, G_NCH, G_CH)
    dst_s = dst.reshape(NS, S_NCH, S_CH)

    x, e = node_feature, edge_feature
    for l in range(2):
        p = params[l]
        w1, w2 = p["W1"], p["W2"]
        w1a, w1b, w1c = w1[:DN], w1[DN:DN + DE], w1[DN + DE:]
        b1 = p["b1"].reshape(1, DH)
        w2a, w2b, w2c = w2[:, :DH], w2[:, DH:DH + DE], w2[:, DH + DE:]
        b2 = p["b2"]
        b2a = b2[:DH].reshape(1, DH)
        b2b = b2[DH:DH + DE].reshape(1, DE)
        b2c = b2[DH + DE:].reshape(1, DH)
        b3 = p["b3"].reshape(1, DH)
        b4 = p["b4"].reshape(1, DN)

        xi, xj = _sc_gather()(x, dst_g, src_g)
        msg2, e = _edge_mlp(xi, xj, e, w1a, w1b, w1c, b1,
                            w2a, b2a, w2b, b2b, w2c, b2c)
        agg = _sc_scatter()(msg2, dst_s)
        x = _node_mlp(agg, p["W3"], b3, p["W4"], b4, relu_out=(l == 0))
    return (x, e)
